# Initial kernel scaffold; baseline (speedup 1.0000x reference)
import jax
import jax.numpy as jnp
from jax.experimental import pallas as pl


def kernel(x, edge_index, batch, W1, b1, g1, be1, W2, b2, Wf1, bf1, gf1, bef1, Wf2, bf2):
    raise NotImplementedError("write your pallas kernel here")


# speedup vs baseline: 8.2606x; 8.2606x over previous
import functools
import jax
import jax.numpy as jnp
from jax import lax
from jax.experimental import pallas as pl
from jax.experimental.pallas import tpu as pltpu
from jax.experimental.pallas import tpu_sc as plsc

_N = 10000
_E = 160000
_G = 64
_D_IN = 256
_D_H = 512

_NP = 10240
_EP = 163840
_DUMMY = _N
_NT = 16
_NC = 2
_RPT = _NP // _NT
_NB = _EP // (_NC * _NT * 128)

_MB = 1280
_GM = _NP // _MB

_mesh = plsc.VectorSubcoreMesh(
    core_axis_name="c", subcore_axis_name="s", num_cores=_NC, num_subcores=_NT)


def _sc_deg_body(dst3, degp, dst_v, ones_v, zv, dacc):
    c = lax.axis_index("c")
    s = lax.axis_index("s")
    wid = c * _NT + s

    @pl.loop(0, 128)
    def _(i):
        ones_v[i, :] = jnp.ones((16,), jnp.float32)

    @pl.loop(0, 64)
    def _(i):
        zv[i, :] = jnp.zeros((16,), jnp.float32)

    for t in range(_RPT // 64):
        pltpu.sync_copy(zv, dacc.at[pl.ds(s * _RPT + t * 64, 64)])
    plsc.subcore_barrier()

    pltpu.sync_copy(dst3.at[wid], dst_v)

    @pl.loop(0, _NB)
    def _(b):
        pltpu.sync_copy(ones_v, dacc.at[dst_v.at[b]], add=True)

    plsc.subcore_barrier()
    for t in range(_RPT // 64):
        pltpu.sync_copy(dacc.at[pl.ds(s * _RPT + t * 64, 64)],
                        degp.at[c, s, pl.ds(t * 64, 64)])


@functools.partial(
    pl.kernel,
    out_type=jax.ShapeDtypeStruct((_NC, _NT, _RPT, 16), jnp.float32),
    mesh=_mesh,
    scratch_types=[
        pltpu.VMEM((_NB, 128), jnp.int32),
        pltpu.VMEM((128, 16), jnp.float32),
        pltpu.VMEM((64, 16), jnp.float32),
        pltpu.VMEM_SHARED((_NP, 16), jnp.float32),
    ],
)
def _sc_deg(dst3, degp, dst_v, ones_v, zv, dacc):
    _sc_deg_body(dst3, degp, dst_v, ones_v, zv, dacc)


def _sc_agg_body(hps, src3, dst3, aggs, src_v, dst_v, gbuf, zv, acc, sem):
    c = lax.axis_index("c")
    s = lax.axis_index("s")
    wid = c * _NT + s

    @pl.loop(0, 64)
    def _(i):
        for k in range(8):
            zv[i, pl.ds(k * 16, 16)] = jnp.zeros((16,), jnp.float32)

    pltpu.sync_copy(src3.at[wid], src_v)
    pltpu.sync_copy(dst3.at[wid], dst_v)

    def run_chunk(hp, agg):
        for t in range(_RPT // 64):
            pltpu.sync_copy(zv, acc.at[pl.ds(s * _RPT + t * 64, 64)])
        plsc.subcore_barrier()

        @pl.loop(0, _NB)
        def _(b):
            pltpu.async_copy(hp.at[src_v.at[b]], gbuf, sem).wait()
            pltpu.sync_copy(gbuf, acc.at[dst_v.at[b]], add=True)

        plsc.subcore_barrier()
        for t in range(_RPT // 64):
            pltpu.sync_copy(acc.at[pl.ds(s * _RPT + t * 64, 64)],
                            agg.at[pl.ds(s * _RPT + t * 64, 64)])
        plsc.subcore_barrier()

    @pl.when(c == 0)
    def _():
        run_chunk(hps[0], aggs[0])
        run_chunk(hps[1], aggs[1])

    @pl.when(c == 1)
    def _():
        run_chunk(hps[2], aggs[2])
        run_chunk(hps[3], aggs[3])


@functools.partial(
    pl.kernel,
    out_type=[jax.ShapeDtypeStruct((_NP, 128), jnp.float32)] * 4,
    mesh=_mesh,
    scratch_types=[
        pltpu.VMEM((_NB, 128), jnp.int32),
        pltpu.VMEM((_NB, 128), jnp.int32),
        pltpu.VMEM((128, 128), jnp.float32),
        pltpu.VMEM((64, 128), jnp.float32),
        pltpu.VMEM_SHARED((_NP, 128), jnp.float32),
        pltpu.SemaphoreType.DMA,
    ],
)
def _sc_agg(hp0, hp1, hp2, hp3, src3, dst3, a0, a1, a2, a3,
            src_v, dst_v, gbuf, zv, acc, sem):
    _sc_agg_body((hp0, hp1, hp2, hp3), src3, dst3, (a0, a1, a2, a3),
                 src_v, dst_v, gbuf, zv, acc, sem)


def _dinv_from(degp_ref):
    dsum = degp_ref[0] + degp_ref[1]
    return lax.rsqrt(dsum[:, 0:1] + 1.0)


def _tc1_body(x_ref, w_ref, degp_ref, h0, h1, h2, h3):
    dinv = _dinv_from(degp_ref)
    h = jnp.dot(x_ref[...], w_ref[...], preferred_element_type=jnp.float32)
    hp = h * dinv
    h0[...] = hp[:, 0:128]
    h1[...] = hp[:, 128:256]
    h2[...] = hp[:, 256:384]
    h3[...] = hp[:, 384:512]


def _tc1(x, w1, degp):
    return pl.pallas_call(
        _tc1_body,
        grid=(_GM,),
        in_specs=[
            pl.BlockSpec((_MB, _D_IN), lambda m: (m, 0)),
            pl.BlockSpec((_D_IN, _D_H), lambda m: (0, 0)),
            pl.BlockSpec((_NC, _MB, 16), lambda m: (0, m, 0)),
        ],
        out_specs=[pl.BlockSpec((_MB, 128), lambda m: (m, 0))] * 4,
        out_shape=[jax.ShapeDtypeStruct((_NP, 128), jnp.float32)] * 4,
    )(x, w1, degp)


def _tc2_body(a0, a1, a2, a3, h0, h1, h2, h3, degp_ref, b_ref,
              out_ref, stats_ref, sum_acc, sq_acc):
    m = pl.program_id(0)

    @pl.when(m == 0)
    def _():
        sum_acc[...] = jnp.zeros((1, _D_H), jnp.float32)
        sq_acc[...] = jnp.zeros((1, _D_H), jnp.float32)

    dinv = _dinv_from(degp_ref)
    agg = jnp.concatenate([a0[...], a1[...], a2[...], a3[...]], axis=1)
    hp = jnp.concatenate([h0[...], h1[...], h2[...], h3[...]], axis=1)
    o = dinv * (agg + hp) + b_ref[...]
    out_ref[...] = o
    rows = m * _MB + lax.broadcasted_iota(jnp.int32, (_MB, 1), 0)
    om = jnp.where(rows < _N, o, 0.0)
    sum_acc[...] += jnp.sum(om, axis=0, keepdims=True)
    sq_acc[...] += jnp.sum(om * om, axis=0, keepdims=True)

    @pl.when(m == _GM - 1)
    def _():
        stats_ref[0:1, :] = sum_acc[...]
        stats_ref[1:2, :] = sq_acc[...]


def _tc2(aggs, hps, degp, b1):
    return pl.pallas_call(
        _tc2_body,
        grid=(_GM,),
        in_specs=[pl.BlockSpec((_MB, 128), lambda m: (m, 0))] * 8 + [
            pl.BlockSpec((_NC, _MB, 16), lambda m: (0, m, 0)),
            pl.BlockSpec((1, _D_H), lambda m: (0, 0)),
        ],
        out_specs=[
            pl.BlockSpec((_MB, _D_H), lambda m: (m, 0)),
            pl.BlockSpec((2, _D_H), lambda m: (0, 0)),
        ],
        out_shape=[
            jax.ShapeDtypeStruct((_NP, _D_H), jnp.float32),
            jax.ShapeDtypeStruct((2, _D_H), jnp.float32),
        ],
        scratch_shapes=[
            pltpu.VMEM((1, _D_H), jnp.float32),
            pltpu.VMEM((1, _D_H), jnp.float32),
        ],
    )(*aggs, *hps, degp, b1)


def _tc3_body(x_ref, stats_ref, g_ref, be_ref, w_ref, degp_ref,
              h0, h1, h2, h3):
    mean = stats_ref[0:1, :] * (1.0 / _N)
    var = stats_ref[1:2, :] * (1.0 / _N) - mean * mean
    xn = g_ref[...] * (x_ref[...] - mean) * lax.rsqrt(var + 1e-5) + be_ref[...]
    a = jnp.maximum(xn, 0.0)
    h = jnp.dot(a, w_ref[...], preferred_element_type=jnp.float32)
    hp = h * _dinv_from(degp_ref)
    h0[...] = hp[:, 0:128]
    h1[...] = hp[:, 128:256]
    h2[...] = hp[:, 256:384]
    h3[...] = hp[:, 384:512]


def _tc3(out1, stats, g1, be1, w2, degp):
    return pl.pallas_call(
        _tc3_body,
        grid=(_GM,),
        in_specs=[
            pl.BlockSpec((_MB, _D_H), lambda m: (m, 0)),
            pl.BlockSpec((2, _D_H), lambda m: (0, 0)),
            pl.BlockSpec((1, _D_H), lambda m: (0, 0)),
            pl.BlockSpec((1, _D_H), lambda m: (0, 0)),
            pl.BlockSpec((_D_H, _D_H), lambda m: (0, 0)),
            pl.BlockSpec((_NC, _MB, 16), lambda m: (0, m, 0)),
        ],
        out_specs=[pl.BlockSpec((_MB, 128), lambda m: (m, 0))] * 4,
        out_shape=[jax.ShapeDtypeStruct((_NP, 128), jnp.float32)] * 4,
    )(out1, stats, g1, be1, w2, degp)


def _tc4_body(a0, a1, a2, a3, h0, h1, h2, h3, degp_ref, b_ref, batch_ref,
              wf1_ref, bf1_ref, gf1_ref, bef1_ref, wf2_ref, bf2_ref,
              out_ref, pooled_acc, cnt_acc):
    m = pl.program_id(0)

    @pl.when(m == 0)
    def _():
        pooled_acc[...] = jnp.zeros((_G, _D_H), jnp.float32)
        cnt_acc[...] = jnp.zeros((1, _G), jnp.float32)

    dinv = _dinv_from(degp_ref)
    agg = jnp.concatenate([a0[...], a1[...], a2[...], a3[...]], axis=1)
    hp = jnp.concatenate([h0[...], h1[...], h2[...], h3[...]], axis=1)
    o = dinv * (agg + hp) + b_ref[...]
    bt = batch_ref[0, 0, :]
    onehot = (bt[:, None] == lax.broadcasted_iota(jnp.int32, (1, _G), 1)
              ).astype(jnp.float32)
    pooled_acc[...] += lax.dot_general(
        onehot, o, (((0,), (0,)), ((), ())),
        preferred_element_type=jnp.float32)
    cnt_acc[...] += jnp.sum(onehot, axis=0, keepdims=True)

    @pl.when(m == _GM - 1)
    def _():
        cnt = jnp.maximum(cnt_acc[0, :], 1.0)
        pooled = pooled_acc[...] / cnt[:, None]
        z = jnp.dot(pooled, wf1_ref[...],
                    preferred_element_type=jnp.float32) + bf1_ref[...]
        mu = jnp.mean(z, axis=0, keepdims=True)
        var = jnp.mean(z * z, axis=0, keepdims=True) - mu * mu
        zn = gf1_ref[...] * (z - mu) * lax.rsqrt(var + 1e-5) + bef1_ref[...]
        r = jnp.maximum(zn, 0.0)
        res = jnp.dot(r, wf2_ref[...],
                      preferred_element_type=jnp.float32) + bf2_ref[...]
        out_ref[...] = res


def _tc4(aggs, hps, degp, b2, batch2, wf1, bf1, gf1, bef1, wf2, bf2):
    return pl.pallas_call(
        _tc4_body,
        grid=(_GM,),
        in_specs=[pl.BlockSpec((_MB, 128), lambda m: (m, 0))] * 8 + [
            pl.BlockSpec((_NC, _MB, 16), lambda m: (0, m, 0)),
            pl.BlockSpec((1, _D_H), lambda m: (0, 0)),
            pl.BlockSpec((1, 1, _MB), lambda m: (m, 0, 0)),
            pl.BlockSpec((_D_H, 128), lambda m: (0, 0)),
            pl.BlockSpec((1, 128), lambda m: (0, 0)),
            pl.BlockSpec((1, 128), lambda m: (0, 0)),
            pl.BlockSpec((1, 128), lambda m: (0, 0)),
            pl.BlockSpec((128, 128), lambda m: (0, 0)),
            pl.BlockSpec((1, 128), lambda m: (0, 0)),
        ],
        out_specs=pl.BlockSpec((_G, 128), lambda m: (0, 0)),
        out_shape=jax.ShapeDtypeStruct((_G, 128), jnp.float32),
        scratch_shapes=[
            pltpu.VMEM((_G, _D_H), jnp.float32),
            pltpu.VMEM((1, _G), jnp.float32),
        ],
    )(*aggs, *hps, degp, b2, batch2, wf1, bf1, gf1, bef1, wf2, bf2)


@jax.jit
def kernel(x, edge_index, batch, W1, b1, g1, be1, W2, b2,
           Wf1, bf1, gf1, bef1, Wf2, bf2):
    src = edge_index[0]
    dst = edge_index[1]
    pad_e = _EP - _E
    src3 = jnp.concatenate(
        [src, jnp.full((pad_e,), _DUMMY, jnp.int32)]).reshape(_NC * _NT, _NB, 128)
    dst3 = jnp.concatenate(
        [dst, jnp.full((pad_e,), _DUMMY, jnp.int32)]).reshape(_NC * _NT, _NB, 128)
    x_pad = jnp.pad(x, ((0, _NP - _N), (0, 0)))
    batch2 = jnp.pad(batch, (0, _NP - _N),
                     constant_values=_G).reshape(_GM, 1, _MB)

    b1r = b1.reshape(1, _D_H)
    b2r = b2.reshape(1, _D_H)
    g1r = g1.reshape(1, _D_H)
    be1r = be1.reshape(1, _D_H)
    wf1p = jnp.pad(Wf1, ((0, 0), (0, 128 - 50)))
    bf1p = jnp.pad(bf1, (0, 128 - 50)).reshape(1, 128)
    gf1p = jnp.pad(gf1, (0, 128 - 50), constant_values=1.0).reshape(1, 128)
    bef1p = jnp.pad(bef1, (0, 128 - 50)).reshape(1, 128)
    wf2p = jnp.pad(Wf2, ((0, 128 - 50), (0, 127)))
    bf2p = jnp.broadcast_to(bf2.reshape(1, 1), (1, 128))

    degp = _sc_deg(dst3).reshape(_NC, _NP, 16)

    hp1 = _tc1(x_pad, W1, degp)
    agg1 = _sc_agg(*hp1, src3, dst3)
    out1, stats = _tc2(agg1, hp1, degp, b1r)
    hp2 = _tc3(out1, stats, g1r, be1r, W2, degp)
    agg2 = _sc_agg(*hp2, src3, dst3)
    out = _tc4(agg2, hp2, degp, b2r, batch2, wf1p, bf1p, gf1p, bef1p,
               wf2p, bf2p)
    return out[:, 0:1]
